# baseline (speedup 1.0000x reference)
import functools

import jax
import jax.numpy as jnp
from jax import lax
from jax.experimental import pallas as pl
from jax.experimental.pallas import tpu as pltpu
from jax.experimental.pallas import tpu_sc as plsc

B, T, V = 16, 50, 100000
T_PAD = 56


def _sc_scores_body(x_hbm, idx_hbm, scores_hbm, idx_v, vals_v, row_v, sem):
    nc = plsc.get_sparse_core_info().num_cores
    wid = lax.axis_index("s") * nc + lax.axis_index("c")

    @pl.when(wid < B)
    def _():
        pltpu.sync_copy(idx_hbm.at[wid], idx_v)
        copies = [
            pltpu.async_copy(x_hbm.at[idx_v.at[k]], vals_v.at[k], sem)
            for k in range(T_PAD * B // 128)
        ]
        for c in copies:
            c.wait()
        acc = jnp.zeros((16,), jnp.float32)
        for t in range(T):
            r, off = t // 8, (t % 8) * 16
            v = vals_v[r, pl.ds(off, 16)]
            acc = acc + jnp.minimum(jnp.maximum(v, -30.0), 30.0)
        row_v[...] = acc
        pltpu.sync_copy(row_v, scores_hbm.at[wid])


@functools.partial(jax.jit, static_argnames=())
def _sc_scores(x_flat, idx):
    mesh = plsc.VectorSubcoreMesh(core_axis_name="c", subcore_axis_name="s")
    kern = functools.partial(
        pl.kernel,
        mesh=mesh,
        out_type=jax.ShapeDtypeStruct((B, B), jnp.float32),
        scratch_types=[
            pltpu.VMEM((T_PAD * B // 128, 128), jnp.int32),
            pltpu.VMEM((T_PAD * B // 128, 128), jnp.float32),
            pltpu.VMEM((16,), jnp.float32),
            pltpu.SemaphoreType.DMA,
        ],
    )(_sc_scores_body)
    return kern(x_flat, idx)


def _combine_body(s_ref, loss_ref, corr_ref):
    s = s_ref[...]
    m = jnp.max(s, axis=1, keepdims=True)
    e = jnp.exp(s - m)
    denom = jnp.log(jnp.sum(e, axis=1, keepdims=True)) + m
    ii = lax.broadcasted_iota(jnp.int32, (B, B), 0)
    jj = lax.broadcasted_iota(jnp.int32, (B, B), 1)
    num = jnp.sum(jnp.where(ii == jj, s, 0.0), axis=1, keepdims=True)
    lt = num - denom
    loss_ref[...] = (-jnp.sum(lt) / (B * T))[None, None]
    corr_ref[...] = (jnp.sum(jnp.exp(lt)) * T)[None, None]


_combine = pl.pallas_call(
    _combine_body,
    out_shape=(
        jax.ShapeDtypeStruct((1, 1), jnp.float32),
        jax.ShapeDtypeStruct((1, 1), jnp.float32),
    ),
)


def kernel(x, Y):
    Y32 = Y.astype(jnp.int32)
    t_off = jnp.arange(T_PAD, dtype=jnp.int32) * V
    t_off = jnp.where(jnp.arange(T_PAD) < T, t_off, 0)
    base = jnp.pad(Y32, ((0, 0), (0, T_PAD - T))) + t_off[None, :]
    j_off = jnp.arange(B, dtype=jnp.int32) * (T * V)
    idx = base[:, :, None] + j_off[None, None, :]
    idx = idx.reshape(B, T_PAD * B // 128, 128)

    scores = _sc_scores(x.reshape(-1), idx)
    loss, corr = _combine(scores)
    return (loss[0, 0], corr[0, 0])


# speedup vs baseline: 150.3623x; 150.3623x over previous
import functools

import jax
import jax.numpy as jnp
from jax import lax
from jax.experimental import pallas as pl
from jax.experimental.pallas import tpu as pltpu
from jax.experimental.pallas import tpu_sc as plsc

B, T, V = 16, 50, 100000
LOOKAHEAD = 10


def _sc_scores_body(x_hbm, y_hbm, scores_hbm, y_v, blk_v, row_v, sem):
    nc = plsc.get_sparse_core_info().num_cores
    wid = lax.axis_index("s") * nc + lax.axis_index("c")

    @pl.when(wid < B)
    def _():
        pltpu.sync_copy(y_hbm.at[wid], y_v)
        lanes = lax.iota(jnp.int32, 16)
        ychunks = [y_v[pl.ds(c * 16, 16)] for c in range(4)]

        yoff = []
        handles = []

        def issue(t):
            c, k = divmod(t, 16)
            yk = jnp.sum(jnp.where(lanes == k, ychunks[c], 0))
            ya = (yk // 128) * 128
            yoff.append(yk - ya)
            handles.append(
                pltpu.async_copy(x_hbm.at[t, :, pl.ds(ya, 128)], blk_v.at[t], sem)
            )

        for t in range(LOOKAHEAD):
            issue(t)
        acc = jnp.zeros((16,), jnp.float32)
        for t in range(T):
            if t + LOOKAHEAD < T:
                issue(t + LOOKAHEAD)
            handles[t].wait()
            col = plsc.load_gather(
                blk_v.at[t], [lanes, jnp.full((16,), yoff[t], jnp.int32)]
            )
            acc = acc + jnp.minimum(jnp.maximum(col, -30.0), 30.0)
        row_v[...] = acc
        pltpu.sync_copy(row_v, scores_hbm.at[wid])


def _sc_scores(xT, yp):
    mesh = plsc.VectorSubcoreMesh(core_axis_name="c", subcore_axis_name="s")
    kern = functools.partial(
        pl.kernel,
        mesh=mesh,
        compiler_params=pltpu.CompilerParams(needs_layout_passes=False),
        out_type=jax.ShapeDtypeStruct((B, B), jnp.float32),
        scratch_types=[
            pltpu.VMEM((64,), jnp.int32),
            pltpu.VMEM((T, 16, 128), jnp.float32),
            pltpu.VMEM((16,), jnp.float32),
            pltpu.SemaphoreType.DMA,
        ],
    )(_sc_scores_body)
    return kern(xT, yp)


def _combine_body(s_ref, loss_ref, corr_ref):
    s = s_ref[...]
    m = jnp.max(s, axis=1, keepdims=True)
    e = jnp.exp(s - m)
    denom = jnp.log(jnp.sum(e, axis=1, keepdims=True)) + m
    ii = lax.broadcasted_iota(jnp.int32, (B, B), 0)
    jj = lax.broadcasted_iota(jnp.int32, (B, B), 1)
    num = jnp.sum(jnp.where(ii == jj, s, 0.0), axis=1, keepdims=True)
    lt = num - denom
    loss_ref[...] = (-jnp.sum(lt) / (B * T))[None, None]
    corr_ref[...] = (jnp.sum(jnp.exp(lt)) * T)[None, None]


_combine = pl.pallas_call(
    _combine_body,
    out_shape=(
        jax.ShapeDtypeStruct((1, 1), jnp.float32),
        jax.ShapeDtypeStruct((1, 1), jnp.float32),
    ),
)


def kernel(x, Y):
    xT = x.transpose(1, 0, 2)
    yp = jnp.pad(Y.astype(jnp.int32), ((0, 0), (0, 64 - T)))
    scores = _sc_scores(xT, yp)
    loss, corr = _combine(scores)
    return (loss[0, 0], corr[0, 0])


# speedup vs baseline: 159.8524x; 1.0631x over previous
import functools

import jax
import jax.numpy as jnp
from jax import lax
from jax.experimental import pallas as pl
from jax.experimental.pallas import tpu as pltpu
from jax.experimental.pallas import tpu_sc as plsc

B, T, V = 16, 50, 100000


def _sc_scores_body(x_hbm, y_hbm, scores_hbm, y_v, blk_v, row_v, sem):
    nc = plsc.get_sparse_core_info().num_cores
    wid = lax.axis_index("s") * nc + lax.axis_index("c")

    @pl.when(wid < B)
    def _():
        pltpu.sync_copy(y_hbm.at[wid], y_v)
        lanes = lax.iota(jnp.int32, 16)
        ycs = [y_v[pl.ds(base, 16)] for base in (0, 16, 32, 34)]

        def extract(t):
            yc = jnp.where(
                t < 16, ycs[0], jnp.where(t < 32, ycs[1], jnp.where(t < 48, ycs[2], ycs[3]))
            )
            base = jnp.where(t < 16, 0, jnp.where(t < 32, 16, jnp.where(t < 48, 32, 34)))
            yk = jnp.sum(jnp.where(lanes == t - base, yc, 0))
            ya = (yk // 128) * 128
            return ya, yk - ya

        def fire(t, carry):
            ya, _ = extract(t)
            pltpu.async_copy(x_hbm.at[t, :, pl.ds(ya, 128)], blk_v.at[t], sem)
            return carry

        lax.fori_loop(0, T, fire, 0, unroll=2)

        def drain(t, acc):
            pltpu.make_async_copy(x_hbm.at[0, :, pl.ds(0, 128)], blk_v.at[0], sem).wait()
            _, ym = extract(t)
            col = plsc.load_gather(
                blk_v, [jnp.full((16,), t, jnp.int32), lanes, jnp.full((16,), ym, jnp.int32)]
            )
            return acc + jnp.minimum(jnp.maximum(col, -30.0), 30.0)

        acc = lax.fori_loop(0, T, drain, jnp.zeros((16,), jnp.float32), unroll=2)
        row_v[...] = acc
        pltpu.sync_copy(row_v, scores_hbm.at[wid])


def _sc_scores(xT, y):
    mesh = plsc.VectorSubcoreMesh(core_axis_name="c", subcore_axis_name="s")
    kern = functools.partial(
        pl.kernel,
        mesh=mesh,
        compiler_params=pltpu.CompilerParams(needs_layout_passes=False),
        out_type=jax.ShapeDtypeStruct((B, B), jnp.float32),
        scratch_types=[
            pltpu.VMEM((T,), jnp.int32),
            pltpu.VMEM((T, 16, 128), jnp.float32),
            pltpu.VMEM((16,), jnp.float32),
            pltpu.SemaphoreType.DMA,
        ],
    )(_sc_scores_body)
    return kern(xT, y)


def _combine_body(s_ref, loss_ref, corr_ref):
    s = s_ref[...]
    m = jnp.max(s, axis=1, keepdims=True)
    e = jnp.exp(s - m)
    denom = jnp.log(jnp.sum(e, axis=1, keepdims=True)) + m
    ii = lax.broadcasted_iota(jnp.int32, (B, B), 0)
    jj = lax.broadcasted_iota(jnp.int32, (B, B), 1)
    num = jnp.sum(jnp.where(ii == jj, s, 0.0), axis=1, keepdims=True)
    lt = num - denom
    loss_ref[...] = (-jnp.sum(lt) / (B * T))[None, None]
    corr_ref[...] = (jnp.sum(jnp.exp(lt)) * T)[None, None]


_combine = pl.pallas_call(
    _combine_body,
    out_shape=(
        jax.ShapeDtypeStruct((1, 1), jnp.float32),
        jax.ShapeDtypeStruct((1, 1), jnp.float32),
    ),
)


def kernel(x, Y):
    xT = x.transpose(1, 0, 2)
    scores = _sc_scores(xT, Y.astype(jnp.int32))
    loss, corr = _combine(scores)
    return (loss[0, 0], corr[0, 0])


# speedup vs baseline: 169.3978x; 1.0597x over previous
import functools

import jax
import jax.numpy as jnp
from jax import lax
from jax.experimental import pallas as pl
from jax.experimental.pallas import tpu as pltpu
from jax.experimental.pallas import tpu_sc as plsc

B, T, V = 16, 50, 100000
TH = T // 2


def _sc_scores_body(x_hbm, y_hbm, scores_hbm, y_v, blk_v, row_v, sem):
    c = lax.axis_index("c")
    s = lax.axis_index("s")
    r = c * 8 + lax.rem(s, 8)
    h = s // 8
    lanes = lax.iota(jnp.int32, 16)

    pltpu.sync_copy(y_hbm.at[r], y_v)
    ycs = [y_v[pl.ds(base, 16)] for base in (0, 16, 32, 34)]

    def extract(t):
        yc = jnp.where(
            t < 16, ycs[0], jnp.where(t < 32, ycs[1], jnp.where(t < 48, ycs[2], ycs[3]))
        )
        base = jnp.where(t < 16, 0, jnp.where(t < 32, 16, jnp.where(t < 48, 32, 34)))
        yk = jnp.sum(jnp.where(lanes == t - base, yc, 0))
        ya = (yk // 128) * 128
        return ya, yk - ya

    t0 = h * TH

    def fire(u, carry):
        ya, _ = extract(t0 + u)
        pltpu.async_copy(x_hbm.at[t0 + u, :, pl.ds(ya, 128)], blk_v.at[u], sem)
        return carry

    lax.fori_loop(0, TH, fire, 0, unroll=2)

    def drain(u, acc):
        pltpu.make_async_copy(x_hbm.at[0, :, pl.ds(0, 128)], blk_v.at[0], sem).wait()
        _, ym = extract(t0 + u)
        col = plsc.load_gather(
            blk_v, [jnp.full((16,), u, jnp.int32), lanes, jnp.full((16,), ym, jnp.int32)]
        )
        return acc + jnp.minimum(jnp.maximum(col, -30.0), 30.0)

    acc = lax.fori_loop(0, TH, drain, jnp.zeros((16,), jnp.float32), unroll=2)
    row_v[...] = acc
    pltpu.sync_copy(row_v, scores_hbm.at[h, r])


def _sc_scores(xT, y):
    mesh = plsc.VectorSubcoreMesh(core_axis_name="c", subcore_axis_name="s")
    kern = functools.partial(
        pl.kernel,
        mesh=mesh,
        compiler_params=pltpu.CompilerParams(needs_layout_passes=False),
        out_type=jax.ShapeDtypeStruct((2, B, B), jnp.float32),
        scratch_types=[
            pltpu.VMEM((T,), jnp.int32),
            pltpu.VMEM((TH, 16, 128), jnp.float32),
            pltpu.VMEM((16,), jnp.float32),
            pltpu.SemaphoreType.DMA,
        ],
    )(_sc_scores_body)
    return kern(xT, y)


def _combine_body(sp_ref, loss_ref, corr_ref):
    s = sp_ref[0] + sp_ref[1]
    m = jnp.max(s, axis=1, keepdims=True)
    e = jnp.exp(s - m)
    denom = jnp.log(jnp.sum(e, axis=1, keepdims=True)) + m
    ii = lax.broadcasted_iota(jnp.int32, (B, B), 0)
    jj = lax.broadcasted_iota(jnp.int32, (B, B), 1)
    num = jnp.sum(jnp.where(ii == jj, s, 0.0), axis=1, keepdims=True)
    lt = num - denom
    loss_ref[...] = (-jnp.sum(lt) / (B * T))[None, None]
    corr_ref[...] = (jnp.sum(jnp.exp(lt)) * T)[None, None]


_combine = pl.pallas_call(
    _combine_body,
    out_shape=(
        jax.ShapeDtypeStruct((1, 1), jnp.float32),
        jax.ShapeDtypeStruct((1, 1), jnp.float32),
    ),
)


def kernel(x, Y):
    xT = x.transpose(1, 0, 2)
    scores_p = _sc_scores(xT, Y.astype(jnp.int32))
    loss, corr = _combine(scores_p)
    return (loss[0, 0], corr[0, 0])
